# Initial kernel scaffold; baseline (speedup 1.0000x reference)
import jax
import jax.numpy as jnp
from jax.experimental import pallas as pl


def kernel(sequence, modifications, seq_table, mod_table):
    raise NotImplementedError("write your pallas kernel here")


# speedup vs baseline: 4.6476x; 4.6476x over previous
import functools

import jax
import jax.numpy as jnp
from jax import lax
from jax.experimental import pallas as pl
from jax.experimental.pallas import tpu as pltpu
from jax.experimental.pallas import tpu_sc as plsc

MOD_VOCAB = 11
OUT_DIM = 80

_CHUNK = 128
_LANES = 16


def _build_sc_call(n_tokens: int):
    info = plsc.get_sparse_core_info()
    nc, ns = info.num_cores, info.num_subcores
    nw = nc * ns
    tpw = n_tokens // nw
    cpw = tpw // _CHUNK
    assert tpw * nw == n_tokens and cpw * _CHUNK == tpw

    mesh = plsc.VectorSubcoreMesh(core_axis_name="c", subcore_axis_name="s")

    @functools.partial(
        pl.kernel,
        mesh=mesh,
        compiler_params=pltpu.CompilerParams(use_tc_tiling_on_sc=False),
        out_type=jax.ShapeDtypeStruct((n_tokens, OUT_DIM), jnp.float32),
        scratch_types=[
            pltpu.VMEM((cpw, _CHUNK), jnp.int32),
            pltpu.VMEM((cpw, _CHUNK), jnp.int32),
            pltpu.VMEM((_CHUNK, OUT_DIM), jnp.float32),
            pltpu.SemaphoreType.DMA,
        ],
    )
    def enc(seq_hbm, mod_hbm, table_hbm, out_hbm, idx_v, mod_v, rows_v, sem):
        wid = lax.axis_index("s") * nc + lax.axis_index("c")
        row0 = wid * cpw

        pltpu.sync_copy(seq_hbm.at[pl.ds(row0, cpw)], idx_v)
        pltpu.sync_copy(mod_hbm.at[pl.ds(row0, cpw)], mod_v)

        def comb_body(j, _):
            for k in range(_CHUNK // _LANES):
                sl = pl.ds(k * _LANES, _LANES)
                idx_v[j, sl] = idx_v[j, sl] * MOD_VOCAB + mod_v[j, sl]
            return 0

        lax.fori_loop(0, cpw, comb_body, 0)

        out_base = wid * tpw

        def gather_body(j, _):
            pltpu.async_copy(table_hbm.at[idx_v.at[j]], rows_v, sem).wait()
            pltpu.sync_copy(
                rows_v, out_hbm.at[pl.ds(out_base + j * _CHUNK, _CHUNK)])
            return 0

        lax.fori_loop(0, cpw, gather_body, 0)

    return enc


def kernel(sequence, modifications, seq_table, mod_table):
    b, l = sequence.shape
    n_tokens = b * l
    aa_vocab = seq_table.shape[0]

    comb_table = jnp.concatenate(
        [
            jnp.repeat(seq_table, MOD_VOCAB, axis=0),
            jnp.tile(mod_table, (aa_vocab, 1)),
        ],
        axis=1,
    )

    seq2d = sequence.reshape(n_tokens // _CHUNK, _CHUNK).astype(jnp.int32)
    mod2d = modifications.reshape(n_tokens // _CHUNK, _CHUNK).astype(jnp.int32)

    out = _build_sc_call(n_tokens)(seq2d, mod2d, comb_table)
    return out.reshape(b, l, OUT_DIM)


# speedup vs baseline: 4.6537x; 1.0013x over previous
import functools

import jax
import jax.numpy as jnp
from jax import lax
from jax.experimental import pallas as pl
from jax.experimental.pallas import tpu as pltpu
from jax.experimental.pallas import tpu_sc as plsc

MOD_VOCAB = 11
OUT_DIM = 80

_CHUNK = 128
_LANES = 16


def _build_sc_call(n_tokens: int):
    info = plsc.get_sparse_core_info()
    nc, ns = info.num_cores, info.num_subcores
    nw = nc * ns
    tpw = n_tokens // nw
    cpw = tpw // _CHUNK
    assert tpw * nw == n_tokens and cpw * _CHUNK == tpw

    nbuf = 4
    n_outer = cpw // nbuf
    assert n_outer * nbuf == cpw and n_outer >= 2

    mesh = plsc.VectorSubcoreMesh(core_axis_name="c", subcore_axis_name="s")

    @functools.partial(
        pl.kernel,
        mesh=mesh,
        compiler_params=pltpu.CompilerParams(use_tc_tiling_on_sc=False),
        out_type=jax.ShapeDtypeStruct((n_tokens, OUT_DIM), jnp.float32),
        scratch_types=[
            pltpu.VMEM((cpw, _CHUNK), jnp.int32),
            pltpu.VMEM((cpw, _CHUNK), jnp.int32),
            [pltpu.VMEM((_CHUNK, OUT_DIM), jnp.float32) for _ in range(nbuf)],
            [pltpu.SemaphoreType.DMA for _ in range(nbuf)],
            [pltpu.SemaphoreType.DMA for _ in range(nbuf)],
        ],
    )
    def enc(seq_hbm, mod_hbm, table_hbm, out_hbm, idx_v, mod_v, rows, gsem, wsem):
        wid = lax.axis_index("s") * nc + lax.axis_index("c")
        row0 = wid * cpw

        pltpu.sync_copy(seq_hbm.at[pl.ds(row0, cpw)], idx_v)
        pltpu.sync_copy(mod_hbm.at[pl.ds(row0, cpw)], mod_v)

        def comb_body(j, _):
            for k in range(_CHUNK // _LANES):
                sl = pl.ds(k * _LANES, _LANES)
                idx_v[j, sl] = idx_v[j, sl] * MOD_VOCAB + mod_v[j, sl]
            return 0

        lax.fori_loop(0, cpw, comb_body, 0)

        out_base = wid * tpw

        def start_gather(b, j):
            pltpu.async_copy(table_hbm.at[idx_v.at[j]], rows[b], gsem[b])

        def wait_gather(b, j):
            pltpu.make_async_copy(
                table_hbm.at[idx_v.at[j]], rows[b], gsem[b]).wait()

        def out_slice(j):
            return out_hbm.at[pl.ds(out_base + j * _CHUNK, _CHUNK)]

        def start_write(b, j):
            pltpu.async_copy(rows[b], out_slice(j), wsem[b])

        def wait_write(b, j):
            pltpu.make_async_copy(rows[b], out_slice(j), wsem[b]).wait()

        for b in range(nbuf):
            start_gather(b, b)

        def outer(t, _):
            j0 = t * nbuf
            for b in range(nbuf):
                wait_gather(b, j0 + b)
                start_write(b, j0 + b)
            for b in range(nbuf):
                wait_write(b, j0 + b)
                start_gather(b, j0 + b + nbuf)
            return 0

        lax.fori_loop(0, n_outer - 1, outer, 0)

        j0 = (n_outer - 1) * nbuf
        for b in range(nbuf):
            wait_gather(b, j0 + b)
            start_write(b, j0 + b)
        for b in range(nbuf):
            wait_write(b, j0 + b)

    return enc


def kernel(sequence, modifications, seq_table, mod_table):
    b, l = sequence.shape
    n_tokens = b * l
    aa_vocab = seq_table.shape[0]

    comb_table = jnp.concatenate(
        [
            jnp.repeat(seq_table, MOD_VOCAB, axis=0),
            jnp.tile(mod_table, (aa_vocab, 1)),
        ],
        axis=1,
    )

    seq2d = sequence.reshape(n_tokens // _CHUNK, _CHUNK).astype(jnp.int32)
    mod2d = modifications.reshape(n_tokens // _CHUNK, _CHUNK).astype(jnp.int32)

    out = _build_sc_call(n_tokens)(seq2d, mod2d, comb_table)
    return out.reshape(b, l, OUT_DIM)


# speedup vs baseline: 6.3823x; 1.3714x over previous
import functools

import jax
import jax.numpy as jnp
from jax import lax
from jax.experimental import pallas as pl
from jax.experimental.pallas import tpu as pltpu
from jax.experimental.pallas import tpu_sc as plsc

MOD_VOCAB = 11
OUT_DIM = 80
_PAD_DIM = 128

_CHUNK = 128
_LANES = 16


def _build_sc_call(n_tokens: int):
    info = plsc.get_sparse_core_info()
    nc, ns = info.num_cores, info.num_subcores
    nw = nc * ns
    tpw = n_tokens // nw
    cpw = tpw // _CHUNK
    assert tpw * nw == n_tokens and cpw * _CHUNK == tpw

    nbuf = 4
    n_outer = cpw // nbuf
    assert n_outer * nbuf == cpw and n_outer >= 2

    mesh = plsc.VectorSubcoreMesh(core_axis_name="c", subcore_axis_name="s")

    @functools.partial(
        pl.kernel,
        mesh=mesh,
        compiler_params=pltpu.CompilerParams(use_tc_tiling_on_sc=False),
        out_type=jax.ShapeDtypeStruct((n_tokens, _PAD_DIM), jnp.float32),
        scratch_types=[
            pltpu.VMEM((cpw, _CHUNK), jnp.int32),
            pltpu.VMEM((cpw, _CHUNK), jnp.int32),
            [pltpu.VMEM((_CHUNK, OUT_DIM), jnp.float32) for _ in range(nbuf)],
            [pltpu.SemaphoreType.DMA for _ in range(nbuf)],
            [pltpu.SemaphoreType.DMA for _ in range(nbuf)],
        ],
    )
    def enc(seq_hbm, mod_hbm, table_hbm, out_hbm, idx_v, mod_v, rows, gsem, wsem):
        wid = lax.axis_index("s") * nc + lax.axis_index("c")
        row0 = wid * cpw

        pltpu.sync_copy(seq_hbm.at[pl.ds(row0, cpw)], idx_v)
        pltpu.sync_copy(mod_hbm.at[pl.ds(row0, cpw)], mod_v)

        def comb_body(j, _):
            for k in range(_CHUNK // _LANES):
                sl = pl.ds(k * _LANES, _LANES)
                idx_v[j, sl] = idx_v[j, sl] * MOD_VOCAB + mod_v[j, sl]
            return 0

        lax.fori_loop(0, cpw, comb_body, 0)

        out_base = wid * tpw

        def start_gather(b, j):
            pltpu.async_copy(table_hbm.at[idx_v.at[j]], rows[b], gsem[b])

        def wait_gather(b, j):
            pltpu.make_async_copy(
                table_hbm.at[idx_v.at[j]], rows[b], gsem[b]).wait()

        def out_slice(j):
            return out_hbm.at[
                pl.ds(out_base + j * _CHUNK, _CHUNK), pl.ds(0, OUT_DIM)]

        def start_write(b, j):
            pltpu.async_copy(rows[b], out_slice(j), wsem[b])

        def wait_write(b, j):
            pltpu.make_async_copy(rows[b], out_slice(j), wsem[b]).wait()

        for b in range(nbuf):
            start_gather(b, b)

        def outer(t, _):
            j0 = t * nbuf
            for b in range(nbuf):
                wait_gather(b, j0 + b)
                start_write(b, j0 + b)
            for b in range(nbuf):
                wait_write(b, j0 + b)
                start_gather(b, j0 + b + nbuf)
            return 0

        lax.fori_loop(0, n_outer - 1, outer, 0)

        j0 = (n_outer - 1) * nbuf
        for b in range(nbuf):
            wait_gather(b, j0 + b)
            start_write(b, j0 + b)
        for b in range(nbuf):
            wait_write(b, j0 + b)

    return enc


def kernel(sequence, modifications, seq_table, mod_table):
    b, l = sequence.shape
    n_tokens = b * l
    aa_vocab = seq_table.shape[0]

    comb_table = jnp.concatenate(
        [
            jnp.repeat(seq_table, MOD_VOCAB, axis=0),
            jnp.tile(mod_table, (aa_vocab, 1)),
        ],
        axis=1,
    )

    seq2d = sequence.reshape(n_tokens // _CHUNK, _CHUNK).astype(jnp.int32)
    mod2d = modifications.reshape(n_tokens // _CHUNK, _CHUNK).astype(jnp.int32)

    out = _build_sc_call(n_tokens)(seq2d, mod2d, comb_table)
    return out[:, :OUT_DIM].reshape(b, l, OUT_DIM)


# speedup vs baseline: 14.5250x; 2.2758x over previous
import functools

import jax
import jax.numpy as jnp
from jax import lax
from jax.experimental import pallas as pl
from jax.experimental.pallas import tpu as pltpu
from jax.experimental.pallas import tpu_sc as plsc

MOD_VOCAB = 11
OUT_DIM = 80
_PAD_DIM = 128

_CHUNK = 128
_LANES = 16


def _build_sc_call(n_tokens: int, n_table_rows: int):
    info = plsc.get_sparse_core_info()
    nc, ns = info.num_cores, info.num_subcores
    nw = nc * ns
    tpw = n_tokens // nw
    cpw = tpw // _CHUNK
    assert tpw * nw == n_tokens and cpw * _CHUNK == tpw

    nbuf = 4
    n_outer = cpw // nbuf
    assert n_outer * nbuf == cpw and n_outer >= 2

    mesh = plsc.VectorSubcoreMesh(core_axis_name="c", subcore_axis_name="s")

    @functools.partial(
        pl.kernel,
        mesh=mesh,
        compiler_params=pltpu.CompilerParams(use_tc_tiling_on_sc=False),
        out_type=jax.ShapeDtypeStruct((n_tokens, _PAD_DIM), jnp.float32),
        scratch_types=[
            pltpu.VMEM((cpw, _CHUNK), jnp.int32),
            pltpu.VMEM((cpw, _CHUNK), jnp.int32),
            [pltpu.VMEM((_CHUNK, OUT_DIM), jnp.float32) for _ in range(nbuf)],
            [pltpu.SemaphoreType.DMA for _ in range(nbuf)],
            [pltpu.SemaphoreType.DMA for _ in range(nbuf)],
            pltpu.VMEM_SHARED((n_table_rows, OUT_DIM), jnp.float32),
        ],
    )
    def enc(seq_hbm, mod_hbm, table_hbm, out_hbm, idx_v, mod_v, rows, gsem,
            wsem, tbl_s):
        sid = lax.axis_index("s")
        wid = sid * nc + lax.axis_index("c")
        row0 = wid * cpw

        @pl.when(sid == 0)
        def _():
            pltpu.sync_copy(table_hbm, tbl_s)

        pltpu.sync_copy(seq_hbm.at[pl.ds(row0, cpw)], idx_v)
        pltpu.sync_copy(mod_hbm.at[pl.ds(row0, cpw)], mod_v)
        plsc.subcore_barrier()

        def comb_body(j, _):
            for k in range(_CHUNK // _LANES):
                sl = pl.ds(k * _LANES, _LANES)
                idx_v[j, sl] = idx_v[j, sl] * MOD_VOCAB + mod_v[j, sl]
            return 0

        lax.fori_loop(0, cpw, comb_body, 0)

        out_base = wid * tpw

        def start_gather(b, j):
            pltpu.async_copy(tbl_s.at[idx_v.at[j]], rows[b], gsem[b])

        def wait_gather(b, j):
            pltpu.make_async_copy(
                tbl_s.at[idx_v.at[j]], rows[b], gsem[b]).wait()

        def out_slice(j):
            return out_hbm.at[
                pl.ds(out_base + j * _CHUNK, _CHUNK), pl.ds(0, OUT_DIM)]

        def start_write(b, j):
            pltpu.async_copy(rows[b], out_slice(j), wsem[b])

        def wait_write(b, j):
            pltpu.make_async_copy(rows[b], out_slice(j), wsem[b]).wait()

        for b in range(nbuf):
            start_gather(b, b)

        def outer(t, _):
            j0 = t * nbuf
            for b in range(nbuf):
                wait_gather(b, j0 + b)
                start_write(b, j0 + b)
            for b in range(nbuf):
                wait_write(b, j0 + b)
                start_gather(b, j0 + b + nbuf)
            return 0

        lax.fori_loop(0, n_outer - 1, outer, 0)

        j0 = (n_outer - 1) * nbuf
        for b in range(nbuf):
            wait_gather(b, j0 + b)
            start_write(b, j0 + b)
        for b in range(nbuf):
            wait_write(b, j0 + b)

    return enc


def kernel(sequence, modifications, seq_table, mod_table):
    b, l = sequence.shape
    n_tokens = b * l
    aa_vocab = seq_table.shape[0]

    comb_table = jnp.concatenate(
        [
            jnp.repeat(seq_table, MOD_VOCAB, axis=0),
            jnp.tile(mod_table, (aa_vocab, 1)),
        ],
        axis=1,
    )

    seq2d = sequence.reshape(n_tokens // _CHUNK, _CHUNK).astype(jnp.int32)
    mod2d = modifications.reshape(n_tokens // _CHUNK, _CHUNK).astype(jnp.int32)

    out = _build_sc_call(n_tokens, comb_table.shape[0])(
        seq2d, mod2d, comb_table)
    return out[:, :OUT_DIM].reshape(b, l, OUT_DIM)
